# Initial kernel scaffold; baseline (speedup 1.0000x reference)
import jax
import jax.numpy as jnp
from jax import lax
from jax.experimental import pallas as pl
from jax.experimental.pallas import tpu as pltpu
from jax.experimental.pallas import tpu_sc as plsc

N = 10000
E = 160000
D = 256
HD = 128
L = 5
G = 64

NC = 2
NS = 16
CH = 128
RPT = N // NS

EPT = E // NS
NF = EPT // CH
RM = EPT - NF * CH

EPC = E // NC
EPTC = EPC // NS
NFC = EPTC // CH
RMC = EPTC - NFC * CH

BN = 400
NT = N // BN

_MESH = plsc.VectorSubcoreMesh(core_axis_name="c", subcore_axis_name="s")


def _spmm_body(hlo, hhi, src, dst, zrows, slo, shi,
               acc, isrc, idst, rows, isrc_r, idst_r, rows_r, sem):
    c = lax.axis_index("c")
    s = lax.axis_index("s")
    r0 = s * RPT
    e0 = s * EPT

    def run(h_ref, out_ref):
        pltpu.sync_copy(zrows, acc.at[pl.ds(r0, RPT)])
        plsc.subcore_barrier()

        def step(j, carry):
            ej = e0 + j * CH
            pltpu.sync_copy(src.at[pl.ds(ej, CH)], isrc)
            pltpu.sync_copy(dst.at[pl.ds(ej, CH)], idst)
            pltpu.async_copy(h_ref.at[isrc], rows, sem).wait()
            pltpu.sync_copy(rows, acc.at[idst], add=True)
            return carry

        lax.fori_loop(0, NF, step, 0)
        er = e0 + NF * CH
        pltpu.sync_copy(src.at[pl.ds(er, RM)], isrc_r)
        pltpu.sync_copy(dst.at[pl.ds(er, RM)], idst_r)
        pltpu.async_copy(h_ref.at[isrc_r], rows_r, sem).wait()
        pltpu.sync_copy(rows_r, acc.at[idst_r], add=True)
        plsc.subcore_barrier()
        pltpu.sync_copy(acc.at[pl.ds(r0, RPT)], out_ref.at[pl.ds(r0, RPT)])

    @pl.when(c == 0)
    def _():
        run(hlo, slo)

    @pl.when(c == 1)
    def _():
        run(hhi, shi)


def _sc_spmm(hlo, hhi, src, dst, zrows):
    return pl.kernel(
        _spmm_body,
        out_type=(
            jax.ShapeDtypeStruct((N, HD), jnp.float32),
            jax.ShapeDtypeStruct((N, HD), jnp.float32),
        ),
        mesh=_MESH,
        scratch_types=[
            pltpu.VMEM_SHARED((N, HD), jnp.float32),
            pltpu.VMEM((CH,), jnp.int32),
            pltpu.VMEM((CH,), jnp.int32),
            pltpu.VMEM((CH, HD), jnp.float32),
            pltpu.VMEM((RM,), jnp.int32),
            pltpu.VMEM((RM,), jnp.int32),
            pltpu.VMEM((RM, HD), jnp.float32),
            pltpu.SemaphoreType.DMA,
        ],
        name="sc_spmm",
    )(hlo, hhi, src, dst, zrows)


def _counts_body(eb, dst, itab, z16, c0out, c1out,
                 acc, ie, idst, rows, ie_r, idst_r, rows_r, sem):
    c = lax.axis_index("c")
    s = lax.axis_index("s")
    r0 = s * RPT
    e0 = c * EPC + s * EPTC

    pltpu.sync_copy(z16, acc.at[pl.ds(r0, RPT)])
    plsc.subcore_barrier()

    def step(j, carry):
        ej = e0 + j * CH
        pltpu.sync_copy(eb.at[pl.ds(ej, CH)], ie)
        pltpu.sync_copy(dst.at[pl.ds(ej, CH)], idst)
        pltpu.async_copy(itab.at[ie], rows, sem).wait()
        pltpu.sync_copy(rows, acc.at[idst], add=True)
        return carry

    lax.fori_loop(0, NFC, step, 0)
    er = e0 + NFC * CH
    pltpu.sync_copy(eb.at[pl.ds(er, RMC)], ie_r)
    pltpu.sync_copy(dst.at[pl.ds(er, RMC)], idst_r)
    pltpu.async_copy(itab.at[ie_r], rows_r, sem).wait()
    pltpu.sync_copy(rows_r, acc.at[idst_r], add=True)
    plsc.subcore_barrier()

    @pl.when(c == 0)
    def _():
        pltpu.sync_copy(acc.at[pl.ds(r0, RPT)], c0out.at[pl.ds(r0, RPT)])

    @pl.when(c == 1)
    def _():
        pltpu.sync_copy(acc.at[pl.ds(r0, RPT)], c1out.at[pl.ds(r0, RPT)])


def _sc_counts(eb, dst, itab, z16):
    return pl.kernel(
        _counts_body,
        out_type=(
            jax.ShapeDtypeStruct((N, 16), jnp.float32),
            jax.ShapeDtypeStruct((N, 16), jnp.float32),
        ),
        mesh=_MESH,
        scratch_types=[
            pltpu.VMEM_SHARED((N, 16), jnp.float32),
            pltpu.VMEM((CH,), jnp.int32),
            pltpu.VMEM((CH,), jnp.int32),
            pltpu.VMEM((CH, 16), jnp.float32),
            pltpu.VMEM((RMC,), jnp.int32),
            pltpu.VMEM((RMC,), jnp.int32),
            pltpu.VMEM((RMC, 16), jnp.float32),
            pltpu.SemaphoreType.DMA,
        ],
        name="sc_counts",
    )(eb, dst, itab, z16)


def _embed_body(x0, x1, e1, e2, hlo, hhi):
    h = jnp.zeros((BN, D), jnp.float32)
    for k in range(3):
        mk = (x0[...] == float(k)).astype(jnp.float32)
        h = h + mk * e1[k:k + 1, :]
    for k in range(3):
        mk = (x1[...] == float(k)).astype(jnp.float32)
        h = h + mk * e2[k:k + 1, :]
    hlo[...] = h[:, :HD]
    hhi[...] = h[:, HD:]


def _tc_embed(x0f, x1f, e1p, e2p):
    return pl.pallas_call(
        _embed_body,
        grid=(NT,),
        in_specs=[
            pl.BlockSpec((BN, 1), lambda i: (i, 0)),
            pl.BlockSpec((BN, 1), lambda i: (i, 0)),
            pl.BlockSpec((8, D), lambda i: (0, 0)),
            pl.BlockSpec((8, D), lambda i: (0, 0)),
        ],
        out_specs=[
            pl.BlockSpec((BN, HD), lambda i: (i, 0)),
            pl.BlockSpec((BN, HD), lambda i: (i, 0)),
        ],
        out_shape=[
            jax.ShapeDtypeStruct((N, HD), jnp.float32),
            jax.ShapeDtypeStruct((N, HD), jnp.float32),
        ],
        name="tc_embed",
    )(x0f, x1f, e1p, e2p)


def _layer_a_body(slo, shi, hlo, hhi, c0, c1, tp, w1, b1, w2, b2,
                  hh_out, stats_out, acc):
    i = pl.program_id(0)
    ct = c0[...] + c1[...]
    ec = jnp.dot(ct, tp[...], preferred_element_type=jnp.float32)
    ec = ec + tp[9:10, :]
    alo = slo[...] + hlo[...] + ec[:, :HD]
    ahi = shi[...] + hhi[...] + ec[:, HD:]
    hid = (jnp.dot(alo, w1[:HD, :], preferred_element_type=jnp.float32)
           + jnp.dot(ahi, w1[HD:, :], preferred_element_type=jnp.float32))
    hid = jnp.maximum(hid + b1[...], 0.0)
    hh = jnp.dot(hid, w2[...], preferred_element_type=jnp.float32) + b2[...]
    hh_out[...] = hh
    s1 = jnp.sum(hh, axis=0, keepdims=True)
    s2 = jnp.sum(hh * hh, axis=0, keepdims=True)

    @pl.when(i == 0)
    def _():
        acc[0:1, :] = s1
        acc[1:2, :] = s2

    @pl.when(i > 0)
    def _():
        acc[0:1, :] = acc[0:1, :] + s1
        acc[1:2, :] = acc[1:2, :] + s2

    stats_out[...] = acc[...]


def _tc_layer_a(slo, shi, hlo, hhi, c0, c1, tp, w1, b1, w2, b2):
    return pl.pallas_call(
        _layer_a_body,
        grid=(NT,),
        in_specs=[
            pl.BlockSpec((BN, HD), lambda i: (i, 0)),
            pl.BlockSpec((BN, HD), lambda i: (i, 0)),
            pl.BlockSpec((BN, HD), lambda i: (i, 0)),
            pl.BlockSpec((BN, HD), lambda i: (i, 0)),
            pl.BlockSpec((BN, 16), lambda i: (i, 0)),
            pl.BlockSpec((BN, 16), lambda i: (i, 0)),
            pl.BlockSpec((16, D), lambda i: (0, 0)),
            pl.BlockSpec((D, 2 * D), lambda i: (0, 0)),
            pl.BlockSpec((1, 2 * D), lambda i: (0, 0)),
            pl.BlockSpec((2 * D, D), lambda i: (0, 0)),
            pl.BlockSpec((1, D), lambda i: (0, 0)),
        ],
        out_specs=[
            pl.BlockSpec((BN, D), lambda i: (i, 0)),
            pl.BlockSpec((2, D), lambda i: (0, 0)),
        ],
        out_shape=[
            jax.ShapeDtypeStruct((N, D), jnp.float32),
            jax.ShapeDtypeStruct((2, D), jnp.float32),
        ],
        scratch_shapes=[pltpu.VMEM((2, D), jnp.float32)],
        name="tc_layer_a",
    )(slo, shi, hlo, hhi, c0, c1, tp, w1, b1, w2, b2)


def _layer_b_body(hh, stats, gamma, beta, hlo_out, hhi_out):
    st = stats[...]
    mu = st[0:1, :] * (1.0 / N)
    var = st[1:2, :] * (1.0 / N) - mu * mu
    inv = gamma[...] / jnp.sqrt(var + 1e-5)
    y = (hh[...] - mu) * inv + beta[...]
    y = jnp.maximum(y, 0.0)
    hlo_out[...] = y[:, :HD]
    hhi_out[...] = y[:, HD:]


def _tc_layer_b(hh, stats, gamma, beta):
    return pl.pallas_call(
        _layer_b_body,
        grid=(NT,),
        in_specs=[
            pl.BlockSpec((BN, D), lambda i: (i, 0)),
            pl.BlockSpec((2, D), lambda i: (0, 0)),
            pl.BlockSpec((1, D), lambda i: (0, 0)),
            pl.BlockSpec((1, D), lambda i: (0, 0)),
        ],
        out_specs=[
            pl.BlockSpec((BN, HD), lambda i: (i, 0)),
            pl.BlockSpec((BN, HD), lambda i: (i, 0)),
        ],
        out_shape=[
            jax.ShapeDtypeStruct((N, HD), jnp.float32),
            jax.ShapeDtypeStruct((N, HD), jnp.float32),
        ],
        name="tc_layer_b",
    )(hh, stats, gamma, beta)


def _final_body(hh, stats, gamma, beta, bf, pw, pb, h_out, pred_out, pacc, cacc):
    i = pl.program_id(0)
    st = stats[...]
    mu = st[0:1, :] * (1.0 / N)
    var = st[1:2, :] * (1.0 / N) - mu * mu
    inv = gamma[...] / jnp.sqrt(var + 1e-5)
    y = (hh[...] - mu) * inv + beta[...]
    h_out[...] = y
    onehot = (bf[...] == lax.broadcasted_iota(jnp.float32, (BN, G), 1)
              ).astype(jnp.float32)
    ps = lax.dot_general(onehot, y, (((0,), (0,)), ((), ())),
                         preferred_element_type=jnp.float32)
    ones = jnp.ones((BN, 1), jnp.float32)
    cs = lax.dot_general(onehot, ones, (((0,), (0,)), ((), ())),
                         preferred_element_type=jnp.float32)

    @pl.when(i == 0)
    def _():
        pacc[...] = ps
        cacc[...] = cs

    @pl.when(i > 0)
    def _():
        pacc[...] = pacc[...] + ps
        cacc[...] = cacc[...] + cs

    @pl.when(i == NT - 1)
    def _():
        pooled = pacc[...] / jnp.maximum(cacc[...], 1.0)
        pred_out[...] = (jnp.dot(pooled, pw[...],
                                 preferred_element_type=jnp.float32)
                         + pb[...])


def _tc_final(hh, stats, gamma, beta, bf, pw, pb):
    return pl.pallas_call(
        _final_body,
        grid=(NT,),
        in_specs=[
            pl.BlockSpec((BN, D), lambda i: (i, 0)),
            pl.BlockSpec((2, D), lambda i: (0, 0)),
            pl.BlockSpec((1, D), lambda i: (0, 0)),
            pl.BlockSpec((1, D), lambda i: (0, 0)),
            pl.BlockSpec((BN, 1), lambda i: (i, 0)),
            pl.BlockSpec((D, 1), lambda i: (0, 0)),
            pl.BlockSpec((1, 1), lambda i: (0, 0)),
        ],
        out_specs=[
            pl.BlockSpec((BN, D), lambda i: (i, 0)),
            pl.BlockSpec((G, 1), lambda i: (0, 0)),
        ],
        out_shape=[
            jax.ShapeDtypeStruct((N, D), jnp.float32),
            jax.ShapeDtypeStruct((G, 1), jnp.float32),
        ],
        scratch_shapes=[
            pltpu.VMEM((G, D), jnp.float32),
            pltpu.VMEM((G, 1), jnp.float32),
        ],
        name="tc_final",
    )(hh, stats, gamma, beta, bf, pw, pb)


def kernel(x, edge_index, edge_attr, batch, params):
    f32 = jnp.float32
    src = edge_index[0]
    dst = edge_index[1]
    eb = edge_attr[:, 0] * 3 + edge_attr[:, 1]

    x0f = x[:, 0].astype(f32).reshape(N, 1)
    x1f = x[:, 1].astype(f32).reshape(N, 1)
    bf = batch.astype(f32).reshape(N, 1)

    zrows = jnp.zeros((RPT, HD), f32)
    z16 = jnp.zeros((RPT, 16), f32)
    itab = jnp.eye(16, dtype=f32)

    e1p = jnp.concatenate([params['x_emb1'][:3], jnp.zeros((5, D), f32)], axis=0)
    e2p = jnp.concatenate([params['x_emb2'][:3], jnp.zeros((5, D), f32)], axis=0)

    hlo, hhi = _tc_embed(x0f, x1f, e1p, e2p)
    c0, c1 = _sc_counts(eb, dst, itab, z16)

    h_final = None
    pred = None
    for l in range(L):
        lp = params['layers'][l]
        t9 = (lp['ee1'][:3, None, :] + lp['ee2'][None, :3, :]).reshape(9, D)
        const = (lp['ee1'][4] + lp['ee2'][0]).reshape(1, D)
        tp = jnp.concatenate([t9, const, jnp.zeros((6, D), f32)], axis=0)
        b1r = lp['b1'].reshape(1, 2 * D)
        b2r = lp['b2'].reshape(1, D)
        gr = lp['gamma'].reshape(1, D)
        br = lp['beta'].reshape(1, D)

        slo, shi = _sc_spmm(hlo, hhi, src, dst, zrows)
        hh, stats = _tc_layer_a(slo, shi, hlo, hhi, c0, c1, tp,
                                lp['W1'], b1r, lp['W2'], b2r)
        if l < L - 1:
            hlo, hhi = _tc_layer_b(hh, stats, gr, br)
        else:
            h_final, pred = _tc_final(hh, stats, gr, br, bf,
                                      params['pred_W'],
                                      params['pred_b'].reshape(1, 1))
    return (pred, h_final)


# speedup vs baseline: 4.6809x; 4.6809x over previous
import jax
import jax.numpy as jnp
from jax import lax
from jax.experimental import pallas as pl
from jax.experimental.pallas import tpu as pltpu
from jax.experimental.pallas import tpu_sc as plsc

N = 10000
E = 160000
D = 256
HD = 128
L = 5
G = 64

NC = 2
NS = 16
CH = 128
RPT = 624
TAIL = N - NS * RPT

EPT = E // NS
NF = EPT // CH
RM = EPT - NF * CH

EPC = E // NC
EPTC = EPC // NS
NFC = EPTC // CH
RMC = EPTC - NFC * CH

BN = 400
NT = N // BN

_MESH = plsc.VectorSubcoreMesh(core_axis_name="c", subcore_axis_name="s")


def _spmm_body(hlo, hhi, src, dst, zrows, slo, shi,
               acc, isrc, idst, rows, isrc_r, idst_r, rows_r, sem):
    c = lax.axis_index("c")
    s = lax.axis_index("s")
    r0 = s * RPT
    e0 = s * EPT

    def run(h_ref, out_ref):
        pltpu.sync_copy(zrows.at[pl.ds(0, RPT)], acc.at[pl.ds(r0, RPT)])

        @pl.when(s == NS - 1)
        def _():
            pltpu.sync_copy(zrows.at[pl.ds(0, TAIL)],
                            acc.at[pl.ds(NS * RPT, TAIL)])

        plsc.subcore_barrier()

        def step(j, carry):
            ej = e0 + j * CH
            pltpu.sync_copy(src.at[pl.ds(ej, CH)], isrc)
            pltpu.sync_copy(dst.at[pl.ds(ej, CH)], idst)
            pltpu.async_copy(h_ref.at[isrc], rows, sem).wait()
            pltpu.sync_copy(rows, acc.at[idst], add=True)
            return carry

        lax.fori_loop(0, NF, step, 0)
        er = e0 + NF * CH
        pltpu.sync_copy(src.at[pl.ds(er, RM)], isrc_r)
        pltpu.sync_copy(dst.at[pl.ds(er, RM)], idst_r)
        pltpu.async_copy(h_ref.at[isrc_r], rows_r, sem).wait()
        pltpu.sync_copy(rows_r, acc.at[idst_r], add=True)
        plsc.subcore_barrier()
        pltpu.sync_copy(acc.at[pl.ds(r0, RPT)], out_ref.at[pl.ds(r0, RPT)])

        @pl.when(s == NS - 1)
        def _():
            pltpu.sync_copy(acc.at[pl.ds(NS * RPT, TAIL)],
                            out_ref.at[pl.ds(NS * RPT, TAIL)])

    @pl.when(c == 0)
    def _():
        run(hlo, slo)

    @pl.when(c == 1)
    def _():
        run(hhi, shi)


def _sc_spmm(hlo, hhi, src, dst, zrows):
    return pl.kernel(
        _spmm_body,
        out_type=(
            jax.ShapeDtypeStruct((N, HD), jnp.float32),
            jax.ShapeDtypeStruct((N, HD), jnp.float32),
        ),
        mesh=_MESH,
        scratch_types=[
            pltpu.VMEM_SHARED((N, HD), jnp.float32),
            pltpu.VMEM((CH,), jnp.int32),
            pltpu.VMEM((CH,), jnp.int32),
            pltpu.VMEM((CH, HD), jnp.float32),
            pltpu.VMEM((RM,), jnp.int32),
            pltpu.VMEM((RM,), jnp.int32),
            pltpu.VMEM((RM, HD), jnp.float32),
            pltpu.SemaphoreType.DMA,
        ],
        name="sc_spmm",
    )(hlo, hhi, src, dst, zrows)


def _counts_body(eb, dst, itab, zrows, c0out, c1out,
                 acc, ie, idst, rows, ie_r, idst_r, rows_r, sem):
    c = lax.axis_index("c")
    s = lax.axis_index("s")
    r0 = s * RPT
    e0 = c * EPC + s * EPTC

    pltpu.sync_copy(zrows.at[pl.ds(0, RPT)], acc.at[pl.ds(r0, RPT)])

    @pl.when(s == NS - 1)
    def _():
        pltpu.sync_copy(zrows.at[pl.ds(0, TAIL)],
                        acc.at[pl.ds(NS * RPT, TAIL)])

    plsc.subcore_barrier()

    def step(j, carry):
        ej = e0 + j * CH
        pltpu.sync_copy(eb.at[pl.ds(ej, CH)], ie)
        pltpu.sync_copy(dst.at[pl.ds(ej, CH)], idst)
        pltpu.async_copy(itab.at[ie], rows, sem).wait()
        pltpu.sync_copy(rows, acc.at[idst], add=True)
        return carry

    lax.fori_loop(0, NFC, step, 0)
    er = e0 + NFC * CH
    pltpu.sync_copy(eb.at[pl.ds(er, RMC)], ie_r)
    pltpu.sync_copy(dst.at[pl.ds(er, RMC)], idst_r)
    pltpu.async_copy(itab.at[ie_r], rows_r, sem).wait()
    pltpu.sync_copy(rows_r, acc.at[idst_r], add=True)
    plsc.subcore_barrier()

    def writeout(out_ref):
        pltpu.sync_copy(acc.at[pl.ds(r0, RPT)], out_ref.at[pl.ds(r0, RPT)])

        @pl.when(s == NS - 1)
        def _():
            pltpu.sync_copy(acc.at[pl.ds(NS * RPT, TAIL)],
                            out_ref.at[pl.ds(NS * RPT, TAIL)])

    @pl.when(c == 0)
    def _():
        writeout(c0out)

    @pl.when(c == 1)
    def _():
        writeout(c1out)


def _sc_counts(eb, dst, itab, zrows):
    return pl.kernel(
        _counts_body,
        out_type=(
            jax.ShapeDtypeStruct((N, HD), jnp.float32),
            jax.ShapeDtypeStruct((N, HD), jnp.float32),
        ),
        mesh=_MESH,
        scratch_types=[
            pltpu.VMEM_SHARED((N, HD), jnp.float32),
            pltpu.VMEM((CH,), jnp.int32),
            pltpu.VMEM((CH,), jnp.int32),
            pltpu.VMEM((CH, HD), jnp.float32),
            pltpu.VMEM((RMC,), jnp.int32),
            pltpu.VMEM((RMC,), jnp.int32),
            pltpu.VMEM((RMC, HD), jnp.float32),
            pltpu.SemaphoreType.DMA,
        ],
        name="sc_counts",
    )(eb, dst, itab, zrows)


def _embed_body(x0, x1, e1, e2, hlo, hhi):
    h = jnp.zeros((BN, D), jnp.float32)
    for k in range(3):
        mk = (x0[...] == float(k)).astype(jnp.float32)
        h = h + mk * e1[k:k + 1, :]
    for k in range(3):
        mk = (x1[...] == float(k)).astype(jnp.float32)
        h = h + mk * e2[k:k + 1, :]
    hlo[...] = h[:, :HD]
    hhi[...] = h[:, HD:]


def _tc_embed(x0f, x1f, e1p, e2p):
    return pl.pallas_call(
        _embed_body,
        grid=(NT,),
        in_specs=[
            pl.BlockSpec((BN, 1), lambda i: (i, 0)),
            pl.BlockSpec((BN, 1), lambda i: (i, 0)),
            pl.BlockSpec((8, D), lambda i: (0, 0)),
            pl.BlockSpec((8, D), lambda i: (0, 0)),
        ],
        out_specs=[
            pl.BlockSpec((BN, HD), lambda i: (i, 0)),
            pl.BlockSpec((BN, HD), lambda i: (i, 0)),
        ],
        out_shape=[
            jax.ShapeDtypeStruct((N, HD), jnp.float32),
            jax.ShapeDtypeStruct((N, HD), jnp.float32),
        ],
        name="tc_embed",
    )(x0f, x1f, e1p, e2p)


def _layer_a_body(slo, shi, hlo, hhi, c0, c1, tp, w1, b1, w2, b2,
                  hh_out, stats_out, acc):
    i = pl.program_id(0)
    ct = c0[...] + c1[...]
    ec = jnp.dot(ct, tp[...], preferred_element_type=jnp.float32,
                 precision=lax.Precision.HIGHEST)
    ec = ec + tp[9:10, :]
    agg = ec + jnp.concatenate([slo[...] + hlo[...], shi[...] + hhi[...]],
                               axis=1)
    hid = jnp.maximum(
        jnp.dot(agg, w1[...], preferred_element_type=jnp.float32) + b1[...],
        0.0)
    hh = jnp.dot(hid, w2[...], preferred_element_type=jnp.float32) + b2[...]
    hh_out[...] = hh
    s1 = jnp.sum(hh, axis=0, keepdims=True)
    s2 = jnp.sum(hh * hh, axis=0, keepdims=True)

    @pl.when(i == 0)
    def _():
        acc[0:1, :] = s1
        acc[1:2, :] = s2

    @pl.when(i > 0)
    def _():
        acc[0:1, :] = acc[0:1, :] + s1
        acc[1:2, :] = acc[1:2, :] + s2

    stats_out[...] = acc[...]


def _tc_layer_a(slo, shi, hlo, hhi, c0, c1, tp, w1, b1, w2, b2):
    return pl.pallas_call(
        _layer_a_body,
        grid=(NT,),
        in_specs=[
            pl.BlockSpec((BN, HD), lambda i: (i, 0)),
            pl.BlockSpec((BN, HD), lambda i: (i, 0)),
            pl.BlockSpec((BN, HD), lambda i: (i, 0)),
            pl.BlockSpec((BN, HD), lambda i: (i, 0)),
            pl.BlockSpec((BN, HD), lambda i: (i, 0)),
            pl.BlockSpec((BN, HD), lambda i: (i, 0)),
            pl.BlockSpec((HD, D), lambda i: (0, 0)),
            pl.BlockSpec((D, 2 * D), lambda i: (0, 0)),
            pl.BlockSpec((1, 2 * D), lambda i: (0, 0)),
            pl.BlockSpec((2 * D, D), lambda i: (0, 0)),
            pl.BlockSpec((1, D), lambda i: (0, 0)),
        ],
        out_specs=[
            pl.BlockSpec((BN, D), lambda i: (i, 0)),
            pl.BlockSpec((2, D), lambda i: (0, 0)),
        ],
        out_shape=[
            jax.ShapeDtypeStruct((N, D), jnp.float32),
            jax.ShapeDtypeStruct((2, D), jnp.float32),
        ],
        scratch_shapes=[pltpu.VMEM((2, D), jnp.float32)],
        name="tc_layer_a",
    )(slo, shi, hlo, hhi, c0, c1, tp, w1, b1, w2, b2)


def _layer_b_body(hh, stats, gamma, beta, hlo_out, hhi_out):
    st = stats[...]
    mu = st[0:1, :] * (1.0 / N)
    var = st[1:2, :] * (1.0 / N) - mu * mu
    y = (hh[...] - mu) / jnp.sqrt(var + 1e-5) * gamma[...] + beta[...]
    y = jnp.maximum(y, 0.0)
    hlo_out[...] = y[:, :HD]
    hhi_out[...] = y[:, HD:]


def _tc_layer_b(hh, stats, gamma, beta):
    return pl.pallas_call(
        _layer_b_body,
        grid=(NT,),
        in_specs=[
            pl.BlockSpec((BN, D), lambda i: (i, 0)),
            pl.BlockSpec((2, D), lambda i: (0, 0)),
            pl.BlockSpec((1, D), lambda i: (0, 0)),
            pl.BlockSpec((1, D), lambda i: (0, 0)),
        ],
        out_specs=[
            pl.BlockSpec((BN, HD), lambda i: (i, 0)),
            pl.BlockSpec((BN, HD), lambda i: (i, 0)),
        ],
        out_shape=[
            jax.ShapeDtypeStruct((N, HD), jnp.float32),
            jax.ShapeDtypeStruct((N, HD), jnp.float32),
        ],
        name="tc_layer_b",
    )(hh, stats, gamma, beta)


def _final_body(hh, stats, gamma, beta, bf, pw, pb, h_out, pred_out, pacc, cacc):
    i = pl.program_id(0)
    st = stats[...]
    mu = st[0:1, :] * (1.0 / N)
    var = st[1:2, :] * (1.0 / N) - mu * mu
    y = (hh[...] - mu) / jnp.sqrt(var + 1e-5) * gamma[...] + beta[...]
    h_out[...] = y
    gids = lax.broadcasted_iota(jnp.int32, (BN, G), 1).astype(jnp.float32)
    onehot = (bf[...] == gids).astype(jnp.float32)
    ps = lax.dot_general(onehot, y, (((0,), (0,)), ((), ())),
                         preferred_element_type=jnp.float32,
                  precision=lax.Precision.HIGHEST)
    ones = jnp.ones((BN, 1), jnp.float32)
    cs = lax.dot_general(onehot, ones, (((0,), (0,)), ((), ())),
                         preferred_element_type=jnp.float32,
                  precision=lax.Precision.HIGHEST)

    @pl.when(i == 0)
    def _():
        pacc[...] = ps
        cacc[...] = cs

    @pl.when(i > 0)
    def _():
        pacc[...] = pacc[...] + ps
        cacc[...] = cacc[...] + cs

    @pl.when(i == NT - 1)
    def _():
        pooled = pacc[...] / jnp.maximum(cacc[...], 1.0)
        pred_out[...] = (jnp.dot(pooled, pw[...],
                                 preferred_element_type=jnp.float32,
                  precision=lax.Precision.HIGHEST)
                         + pb[...])


def _tc_final(hh, stats, gamma, beta, bf, pw, pb):
    return pl.pallas_call(
        _final_body,
        grid=(NT,),
        in_specs=[
            pl.BlockSpec((BN, D), lambda i: (i, 0)),
            pl.BlockSpec((2, D), lambda i: (0, 0)),
            pl.BlockSpec((1, D), lambda i: (0, 0)),
            pl.BlockSpec((1, D), lambda i: (0, 0)),
            pl.BlockSpec((BN, 1), lambda i: (i, 0)),
            pl.BlockSpec((D, 1), lambda i: (0, 0)),
            pl.BlockSpec((1, 1), lambda i: (0, 0)),
        ],
        out_specs=[
            pl.BlockSpec((BN, D), lambda i: (i, 0)),
            pl.BlockSpec((G, 1), lambda i: (0, 0)),
        ],
        out_shape=[
            jax.ShapeDtypeStruct((N, D), jnp.float32),
            jax.ShapeDtypeStruct((G, 1), jnp.float32),
        ],
        scratch_shapes=[
            pltpu.VMEM((G, D), jnp.float32),
            pltpu.VMEM((G, 1), jnp.float32),
        ],
        name="tc_final",
    )(hh, stats, gamma, beta, bf, pw, pb)


def kernel(x, edge_index, edge_attr, batch, params):
    f32 = jnp.float32
    src = edge_index[0]
    dst = edge_index[1]
    eb = edge_attr[:, 0] * 3 + edge_attr[:, 1]

    x0f = x[:, 0].astype(f32).reshape(N, 1)
    x1f = x[:, 1].astype(f32).reshape(N, 1)
    bf = batch.astype(f32).reshape(N, 1)

    zrows = jnp.zeros((RPT, HD), f32)
    itab = jnp.eye(16, HD, dtype=f32)

    e1p = jnp.concatenate([params['x_emb1'][:3], jnp.zeros((5, D), f32)], axis=0)
    e2p = jnp.concatenate([params['x_emb2'][:3], jnp.zeros((5, D), f32)], axis=0)

    hlo, hhi = _tc_embed(x0f, x1f, e1p, e2p)
    c0, c1 = _sc_counts(eb, dst, itab, zrows)

    h_final = None
    pred = None
    for l in range(L):
        lp = params['layers'][l]
        t9 = (lp['ee1'][:3, None, :] + lp['ee2'][None, :3, :]).reshape(9, D)
        const = (lp['ee1'][4] + lp['ee2'][0]).reshape(1, D)
        tp = jnp.concatenate([t9, const, jnp.zeros((HD - 10, D), f32)], axis=0)
        b1r = lp['b1'].reshape(1, 2 * D)
        b2r = lp['b2'].reshape(1, D)
        gr = lp['gamma'].reshape(1, D)
        br = lp['beta'].reshape(1, D)

        slo, shi = _sc_spmm(hlo, hhi, src, dst, zrows)
        hh, stats = _tc_layer_a(slo, shi, hlo, hhi, c0, c1, tp,
                                lp['W1'], b1r, lp['W2'], b2r)
        if l < L - 1:
            hlo, hhi = _tc_layer_b(hh, stats, gr, br)
        else:
            h_final, pred = _tc_final(hh, stats, gr, br, bf,
                                      params['pred_W'],
                                      params['pred_b'].reshape(1, 1))
    return (pred, h_final)
